# baseline (speedup 1.0000x reference)
import functools

import jax
import jax.numpy as jnp
from jax import lax
from jax.experimental import pallas as pl
from jax.experimental.pallas import tpu as pltpu
from jax.experimental.pallas import tpu_sc as plsc

_NCH = 192
_B = 4
_ROWS = _B * _NCH
_H = 224
_W = 224
_NC = 2
_NS = 16
_NW = _NC * _NS
_RPW = _ROWS // _NW
_IDXPAD = 32
_HSPLIT = 2
_HH = _H // _HSPLIT
_NBUF = 4


def _make_sc_shuffle():
    mesh = plsc.VectorSubcoreMesh(core_axis_name="c", subcore_axis_name="s")

    @functools.partial(
        pl.kernel,
        out_type=jax.ShapeDtypeStruct((_ROWS, _H, _W), jnp.float32),
        mesh=mesh,
        compiler_params=pltpu.CompilerParams(needs_layout_passes=False),
        scratch_types=[
            pltpu.VMEM((_IDXPAD,), jnp.int32),
            pltpu.SemaphoreType.DMA,
        ],
    )
    def shuffle(x_hbm, ids_hbm, out_hbm, idx_v, sem):
        wid = lax.axis_index("s") * _NC + lax.axis_index("c")
        base = wid * _RPW
        pltpu.sync_copy(ids_hbm.at[wid], idx_v)
        c0 = idx_v[pl.ds(0, 16)]
        c1 = idx_v[pl.ds(16, 16)]
        cps = []
        for j in range(_RPW):
            rid = (c0 if j < 16 else c1)[j % 16]
            cps.append(pltpu.async_copy(
                x_hbm.at[pl.ds(rid, 1)],
                out_hbm.at[pl.ds(base + j, 1)], sem))
        for cp in cps:
            cp.wait()

    return shuffle


_sc_shuffle = _make_sc_shuffle()


def kernel(x_list, objective, indices):
    x3 = x_list.reshape(_ROWS, _H, _W)
    src = (jnp.arange(_B, dtype=jnp.int32)[:, None] * _NCH
           + indices[None, :].astype(jnp.int32)).reshape(_NW, _RPW)
    ids = jnp.pad(src, ((0, 0), (0, _IDXPAD - _RPW)))
    y3 = _sc_shuffle(x3, ids)
    return (y3.reshape(_B, _NCH, _H, _W), objective)


# speedup vs baseline: 37.1347x; 37.1347x over previous
import functools

import jax
import jax.numpy as jnp
from jax import lax
from jax.experimental import pallas as pl
from jax.experimental.pallas import tpu as pltpu
from jax.experimental.pallas import tpu_sc as plsc

_NCH = 192
_B = 4
_ROWS = _B * _NCH
_H = 224
_W = 224
_NC = 2
_NS = 16
_NW = _NC * _NS
_RPW = _ROWS // _NW
_IDXPAD = 32
_HSPLIT = 2
_HH = _H // _HSPLIT
_NBUF = 4


def _make_sc_shuffle():
    mesh = plsc.VectorSubcoreMesh(core_axis_name="c", subcore_axis_name="s")

    @functools.partial(
        pl.kernel,
        out_type=jax.ShapeDtypeStruct((_ROWS, _H, _W), jnp.float32),
        mesh=mesh,
        compiler_params=pltpu.CompilerParams(needs_layout_passes=False),
        scratch_types=[
            pltpu.VMEM((_IDXPAD,), jnp.int32),
            pltpu.VMEM((1, _H, _W), jnp.float32),
            pltpu.VMEM((1, _H, _W), jnp.float32),
            pltpu.SemaphoreType.DMA,
            pltpu.SemaphoreType.DMA,
            pltpu.SemaphoreType.DMA,
            pltpu.SemaphoreType.DMA,
        ],
    )
    def shuffle(x_hbm, ids_hbm, out_hbm, idx_v, buf0, buf1, g0, g1, s0, s1):
        wid = lax.axis_index("s") * _NC + lax.axis_index("c")
        base = wid * _RPW
        boff = (wid // (_NCH // _RPW)) * _NCH
        cbase = (wid % (_NCH // _RPW)) * _RPW
        pltpu.sync_copy(ids_hbm.at[pl.ds(cbase, _RPW)],
                        idx_v.at[pl.ds(0, _RPW)])
        c0 = idx_v[pl.ds(0, 16)] + boff
        c1 = idx_v[pl.ds(16, 16)] + boff
        bufs = (buf0, buf1)
        gsems = (g0, g1)
        ssems = (s0, s1)
        gcp = [None, None]
        for j in range(_RPW):
            b = j % 2
            rid = (c0 if j < 16 else c1)[j % 16]
            if j >= 2:
                pltpu.make_async_copy(
                    bufs[b], out_hbm.at[pl.ds(0, 1)], ssems[b]).wait()
            gcp[b] = pltpu.async_copy(
                x_hbm.at[pl.ds(rid, 1)], bufs[b], gsems[b])
            if j >= 1:
                pb = (j - 1) % 2
                gcp[pb].wait()
                pltpu.async_copy(
                    bufs[pb], out_hbm.at[pl.ds(base + j - 1, 1)], ssems[pb])
        lb = (_RPW - 1) % 2
        gcp[lb].wait()
        pltpu.async_copy(
            bufs[lb], out_hbm.at[pl.ds(base + _RPW - 1, 1)], ssems[lb])
        pltpu.make_async_copy(buf0, out_hbm.at[pl.ds(0, 1)], s0).wait()
        pltpu.make_async_copy(buf1, out_hbm.at[pl.ds(0, 1)], s1).wait()

    return shuffle


_sc_shuffle = _make_sc_shuffle()


def kernel(x_list, objective, indices):
    x3 = x_list.reshape(_ROWS, _H, _W)
    y3 = _sc_shuffle(x3, indices.astype(jnp.int32))
    return (y3.reshape(_B, _NCH, _H, _W), objective)
